# baseline (speedup 1.0000x reference)
import functools

import jax
import jax.numpy as jnp
import numpy as np
from jax import lax
from jax.experimental import pallas as pl
from jax.experimental.pallas import tpu as pltpu
from jax.experimental.pallas import tpu_sc as plsc

NC, NS, L = 2, 16, 16
NW = NC * NS
NT = 8192
NCOL = 1536
CPW = NCOL // NW
CHUNK = NT // L
NVEC = NT // L
RADIX = 256
HIST = RADIX * L
MININT = np.int32(-2147483648)


def _transpose_body(x_ref, o_ref):
    o_ref[...] = x_ref[...].T


def _transpose(x):
    bt, bc = 1024, 512
    return pl.pallas_call(
        _transpose_body,
        grid=(NT // bt, NCOL // bc),
        in_specs=[pl.BlockSpec((bt, bc), lambda i, j: (i, j))],
        out_specs=pl.BlockSpec((bc, bt), lambda i, j: (j, i)),
        out_shape=jax.ShapeDtypeStruct((NCOL, NT), jnp.float32),
    )(x)


def _sc_body(pred_hbm, obs_hbm, out_hbm, land, ap, ao, bscr, hist, tot, dbase, acc):
    cid = lax.axis_index("c")
    sid = lax.axis_index("s")
    wid = sid * NC + cid
    lane = lax.iota(jnp.int32, L)
    ones = jnp.ones((L,), jnp.int32)
    zi = jnp.zeros((L,), jnp.int32)
    gidx0 = lane * CHUNK
    lane0 = lane == 0

    acc[...] = jnp.zeros((L,), jnp.float32)

    def sort_column(dst):
        def tbody(i, _):
            y = lax.bitcast_convert_type(land[pl.ds(i * L, L)], jnp.int32)
            dst[pl.ds(i * L, L)] = y ^ (MININT | (y >> 31))
            return 0

        lax.fori_loop(0, NVEC, tbody, 0)

        for t in range(4):
            src = dst if t % 2 == 0 else bscr
            dd = bscr if t % 2 == 0 else dst
            sh = jnp.int32(8 * t)

            def zbody(i, _):
                hist[pl.ds(i * L, L)] = zi
                return 0

            lax.fori_loop(0, HIST // L, zbody, 0)

            def hbody(i, _):
                k = plsc.load_gather(src, [gidx0 + i])
                d = lax.shift_right_logical(k, sh) & 255
                plsc.addupdate_scatter(hist, [(d << 4) | lane], ones)
                return 0

            lax.fori_loop(0, NVEC, hbody, 0)

            def s1body(g, _):
                v = hist[pl.ds(g * L, L)]
                s = plsc.cumsum(v)
                hist[pl.ds(g * L, L)] = s - v
                plsc.store_scatter(tot, [g + zi], jnp.sum(v) + zi, mask=lane0)
                return 0

            lax.fori_loop(0, RADIX, s1body, 0)

            def s2body(g, c):
                v = tot[pl.ds(g * L, L)]
                s = plsc.cumsum(v)
                dbase[pl.ds(g * L, L)] = (s - v) + c
                return c + jnp.sum(v)

            lax.fori_loop(0, RADIX // L, s2body, jnp.int32(0))

            def s3body(g, _):
                bvec = plsc.load_gather(dbase, [g + zi])
                hist[pl.ds(g * L, L)] = hist[pl.ds(g * L, L)] + bvec
                return 0

            lax.fori_loop(0, RADIX, s3body, 0)

            def pbody(i, _):
                k = plsc.load_gather(src, [gidx0 + i])
                d = lax.shift_right_logical(k, sh) & 255
                hidx = (d << 4) | lane
                pos = plsc.load_gather(hist, [hidx])
                plsc.store_scatter(hist, [hidx], pos + ones)
                plsc.store_scatter(dd, [pos], k)
                return 0

            lax.fori_loop(0, NVEC, pbody, 0)

    def col_body(c, _):
        col = wid * CPW + c
        pltpu.sync_copy(pred_hbm.at[col], land)
        sort_column(ap)
        pltpu.sync_copy(obs_hbm.at[col], land)
        sort_column(ao)

        def dbody(i, _):
            ya = ap[pl.ds(i * L, L)]
            yb = ao[pl.ds(i * L, L)]
            xa = ya ^ (MININT | ~(ya >> 31))
            xb = yb ^ (MININT | ~(yb >> 31))
            fa = lax.bitcast_convert_type(xa, jnp.float32)
            fb = lax.bitcast_convert_type(xb, jnp.float32)
            acc[...] = acc[...] + jnp.abs(fa - fb)
            return 0

        lax.fori_loop(0, NVEC, dbody, 0)
        return 0

    lax.fori_loop(0, CPW, col_body, 0)
    pltpu.sync_copy(acc, out_hbm.at[wid])


_sc_wasserstein = functools.partial(
    pl.kernel,
    out_type=jax.ShapeDtypeStruct((NW, L), jnp.float32),
    mesh=plsc.VectorSubcoreMesh(core_axis_name="c", subcore_axis_name="s"),
    compiler_params=pltpu.CompilerParams(needs_layout_passes=False),
    scratch_types=[
        pltpu.VMEM((NT,), jnp.float32),
        pltpu.VMEM((NT,), jnp.int32),
        pltpu.VMEM((NT,), jnp.int32),
        pltpu.VMEM((NT,), jnp.int32),
        pltpu.VMEM((HIST,), jnp.int32),
        pltpu.VMEM((RADIX,), jnp.int32),
        pltpu.VMEM((RADIX,), jnp.int32),
        pltpu.VMEM((L,), jnp.float32),
    ],
)(_sc_body)


def kernel(pred_waveforms, obs_waveforms):
    nt, ntr, ch = pred_waveforms.shape
    pred_t = _transpose(pred_waveforms.reshape(nt, ntr * ch))
    obs_t = _transpose(obs_waveforms.reshape(nt, ntr * ch))
    partials = _sc_wasserstein(pred_t, obs_t)
    return jnp.sum(partials) / (nt * ntr * ch)


# speedup vs baseline: 1.7949x; 1.7949x over previous
import functools

import jax
import jax.numpy as jnp
import numpy as np
from jax import lax
from jax.experimental import pallas as pl
from jax.experimental.pallas import tpu as pltpu
from jax.experimental.pallas import tpu_sc as plsc

NC, NS, L = 2, 16, 16
NW = NC * NS
NT = 8192
NCOL = 1536
CPW = NCOL // NW
CHUNK = NT // L
NVEC = NT // L
RADIX = 256
HIST = RADIX * L
MININT = np.int32(-2147483648)


def _transpose_body(x_ref, o_ref):
    o_ref[...] = x_ref[...].T


def _transpose(x):
    bt, bc = 1024, 512
    return pl.pallas_call(
        _transpose_body,
        grid=(NT // bt, NCOL // bc),
        in_specs=[pl.BlockSpec((bt, bc), lambda i, j: (i, j))],
        out_specs=pl.BlockSpec((bc, bt), lambda i, j: (j, i)),
        out_shape=jax.ShapeDtypeStruct((NCOL, NT), jnp.float32),
    )(x)


def _sc_body(pred_hbm, obs_hbm, out_hbm,
             land0, land1,
             a0, a1, a2, a3, b0, b1, b2, b3,
             hx0, hx1, hx2, hx3, hy0, hy1, hy2, hy3,
             tt0, tt1, tt2, tt3, dg0, dg1, dg2, dg3,
             acc0, acc1, acc2, acc3, sem0, sem1):
    cid = lax.axis_index("c")
    sid = lax.axis_index("s")
    wid = sid * NC + cid
    lane = lax.iota(jnp.int32, L)
    ones = jnp.ones((L,), jnp.int32)
    zi = jnp.zeros((L,), jnp.int32)
    lanebase = lane * RADIX
    A = [a0, a1, a2, a3]
    B = [b0, b1, b2, b3]
    HX = [hx0, hx1, hx2, hx3]
    HY = [hy0, hy1, hy2, hy3]
    TOT = [tt0, tt1, tt2, tt3]
    DB = [dg0, dg1, dg2, dg3]
    ACC = [acc0, acc1, acc2, acc3]
    LANDS = [land0, land1]
    SEMS = [sem0, sem1]

    for s in range(4):
        ACC[s][...] = jnp.zeros((L,), jnp.float32)

    def zero_init(i, _):
        for s in range(4):
            HX[s][pl.ds(i * L, L)] = zi
        return 0

    lax.fori_loop(0, HIST // L, zero_init, 0, unroll=2)

    def pair_step(p, _):
        c0 = wid * CPW + 2 * p
        srcs = [(pred_hbm, c0), (obs_hbm, c0), (pred_hbm, c0 + 1), (obs_hbm, c0 + 1)]

        def dmad(s):
            ref, col = srcs[s]
            return pltpu.make_async_copy(ref.at[col], LANDS[s % 2], SEMS[s % 2])

        dmad(0).start()
        for s in range(4):
            if s < 3:
                dmad(s + 1).start()
            dmad(s).wait()
            land = LANDS[s % 2]

            def tbody(i, _):
                y = lax.bitcast_convert_type(land[pl.ds(i * L, L)], jnp.int32)
                m = y ^ (MININT | (y >> 31))
                A[s][pl.ds(i * L, L)] = m
                plsc.addupdate_scatter(HX[s], [lanebase + (m & 255)], ones)
                return 0

            lax.fori_loop(0, NVEC, tbody, 0, unroll=4)

        for t in range(4):
            cur = HX if t % 2 == 0 else HY
            nxt = HY if t % 2 == 0 else HX
            src = A if t % 2 == 0 else B
            dst = B if t % 2 == 0 else A

            if t > 0:
                def hbody(i, _):
                    for s in range(4):
                        k = src[s][pl.ds(i * L, L)]
                        d = lax.shift_right_logical(k, 8 * t)
                        if t < 3:
                            d = d & 255
                        plsc.addupdate_scatter(cur[s], [lanebase + d], ones)
                    return 0

                lax.fori_loop(0, NVEC, hbody, 0, unroll=2)

            def l1(g, _):
                for s in range(4):
                    vs = [cur[s][pl.ds(l * RADIX + g * L, L)] for l in range(L)]
                    while len(vs) > 1:
                        vs = [vs[2 * j] + vs[2 * j + 1] for j in range(len(vs) // 2)]
                    TOT[s][pl.ds(g * L, L)] = vs[0]
                return 0

            lax.fori_loop(0, RADIX // L, l1, 0)

            def l2(g, carry):
                nc = []
                for s in range(4):
                    v = TOT[s][pl.ds(g * L, L)]
                    sc = plsc.cumsum(v)
                    DB[s][pl.ds(g * L, L)] = (sc - v) + carry[s]
                    nc.append(carry[s] + jnp.sum(v))
                return tuple(nc)

            lax.fori_loop(0, RADIX // L, l2, (jnp.int32(0),) * 4)

            def l3(g, _):
                for s in range(4):
                    run = DB[s][pl.ds(g * L, L)]
                    for l in range(L):
                        v = cur[s][pl.ds(l * RADIX + g * L, L)]
                        cur[s][pl.ds(l * RADIX + g * L, L)] = run
                        nxt[s][pl.ds(l * RADIX + g * L, L)] = zi
                        run = run + v
                return 0

            lax.fori_loop(0, RADIX // L, l3, 0)

            def pbody(i, _):
                for s in range(4):
                    k = src[s][pl.ds(i * L, L)]
                    if t == 0:
                        d = k & 255
                    elif t == 3:
                        d = lax.shift_right_logical(k, 24)
                    else:
                        d = lax.shift_right_logical(k, 8 * t) & 255
                    hidx = lanebase + d
                    r = plsc.load_gather(cur[s], [hidx])
                    plsc.store_scatter(cur[s], [hidx], r + ones)
                    pos = ((r & (CHUNK - 1)) << 4) | (r >> 9)
                    plsc.store_scatter(dst[s], [pos], k)
                return 0

            lax.fori_loop(0, NVEC, pbody, 0, unroll=2)

        def dbody(i, _):
            for pi, (sa, sb) in enumerate(((0, 1), (2, 3))):
                for par in range(2):
                    idx = 2 * i + par
                    ya = A[sa][pl.ds(idx * L, L)]
                    yb = A[sb][pl.ds(idx * L, L)]
                    xa = ya ^ (MININT | ~(ya >> 31))
                    xb = yb ^ (MININT | ~(yb >> 31))
                    fa = lax.bitcast_convert_type(xa, jnp.float32)
                    fb = lax.bitcast_convert_type(xb, jnp.float32)
                    ACC[2 * pi + par][...] = ACC[2 * pi + par][...] + jnp.abs(fa - fb)
            return 0

        lax.fori_loop(0, NVEC // 2, dbody, 0)
        return 0

    lax.fori_loop(0, CPW // 2, pair_step, 0)
    acc0[...] = (ACC[0][...] + ACC[1][...]) + (ACC[2][...] + ACC[3][...])
    pltpu.sync_copy(acc0, out_hbm.at[wid])


_sc_wasserstein = functools.partial(
    pl.kernel,
    out_type=jax.ShapeDtypeStruct((NW, L), jnp.float32),
    mesh=plsc.VectorSubcoreMesh(core_axis_name="c", subcore_axis_name="s"),
    compiler_params=pltpu.CompilerParams(needs_layout_passes=False),
    scratch_types=(
        [pltpu.VMEM((NT,), jnp.float32) for _ in range(2)]
        + [pltpu.VMEM((NT,), jnp.int32) for _ in range(8)]
        + [pltpu.VMEM((HIST,), jnp.int32) for _ in range(8)]
        + [pltpu.VMEM((RADIX,), jnp.int32) for _ in range(8)]
        + [pltpu.VMEM((L,), jnp.float32) for _ in range(4)]
        + [pltpu.SemaphoreType.DMA for _ in range(2)]
    ),
)(_sc_body)


def kernel(pred_waveforms, obs_waveforms):
    nt, ntr, ch = pred_waveforms.shape
    pred_t = _transpose(pred_waveforms.reshape(nt, ntr * ch))
    obs_t = _transpose(obs_waveforms.reshape(nt, ntr * ch))
    partials = _sc_wasserstein(pred_t, obs_t)
    return jnp.sum(partials) / (nt * ntr * ch)


# speedup vs baseline: 4.4035x; 2.4533x over previous
import functools

import jax
import jax.numpy as jnp
import numpy as np
from jax import lax
from jax.experimental import pallas as pl
from jax.experimental.pallas import tpu as pltpu
from jax.experimental.pallas import tpu_sc as plsc

NC, NS, L = 2, 16, 16
NW = NC * NS
NT = 8192
NCOL = 1536
CPW = NCOL // NW
CHUNK = NT // L
NVEC = NT // L
RADIX = 256
HIST = RADIX * L
MININT = np.int32(-2147483648)


def _transpose_body(x_ref, o_ref):
    o_ref[...] = x_ref[...].T


def _transpose(x):
    bt, bc = 1024, 512
    return pl.pallas_call(
        _transpose_body,
        grid=(NT // bt, NCOL // bc),
        in_specs=[pl.BlockSpec((bt, bc), lambda i, j: (i, j))],
        out_specs=pl.BlockSpec((bc, bt), lambda i, j: (j, i)),
        out_shape=jax.ShapeDtypeStruct((NCOL, NT), jnp.float32),
    )(x)


def _sc_body(pred_hbm, obs_hbm, out_hbm,
             land0, land1,
             a0, a1, a2, a3, b0, b1, b2, b3,
             hx0, hx1, hx2, hx3, hy0, hy1, hy2, hy3,
             tt0, tt1, tt2, tt3, dg0, dg1, dg2, dg3,
             acc0, acc1, acc2, acc3, sem0, sem1):
    cid = lax.axis_index("c")
    sid = lax.axis_index("s")
    wid = sid * NC + cid
    lane = lax.iota(jnp.int32, L)
    ones = jnp.ones((L,), jnp.int32)
    zi = jnp.zeros((L,), jnp.int32)
    lanebase = lane * RADIX
    A = [a0, a1, a2, a3]
    B = [b0, b1, b2, b3]
    HX = [hx0, hx1, hx2, hx3]
    HY = [hy0, hy1, hy2, hy3]
    TOT = [tt0, tt1, tt2, tt3]
    DB = [dg0, dg1, dg2, dg3]
    ACC = [acc0, acc1, acc2, acc3]
    LANDS = [land0, land1]
    SEMS = [sem0, sem1]

    for s in range(4):
        ACC[s][...] = jnp.zeros((L,), jnp.float32)

    def zero_init(i, _):
        for s in range(4):
            HX[s][pl.ds(i * L, L)] = zi
        return 0

    lax.fori_loop(0, HIST // L, zero_init, 0, unroll=2)

    def pair_step(p, _):
        c0 = wid * CPW + 2 * p
        srcs = [(pred_hbm, c0), (obs_hbm, c0), (pred_hbm, c0 + 1), (obs_hbm, c0 + 1)]

        def dmad(s):
            ref, col = srcs[s]
            return pltpu.make_async_copy(ref.at[col], LANDS[s % 2], SEMS[s % 2])

        dmad(0).start()
        for s in range(4):
            if s < 3:
                dmad(s + 1).start()
            dmad(s).wait()
            land = LANDS[s % 2]

            def tbody(j, _):
                i0 = 4 * j
                ys = [lax.bitcast_convert_type(land[pl.ds((i0 + u) * L, L)], jnp.int32)
                      for u in range(4)]
                ms = [y ^ (MININT | (y >> 31)) for y in ys]
                for u in range(4):
                    A[s][pl.ds((i0 + u) * L, L)] = ms[u]
                for u in range(4):
                    plsc.addupdate_scatter(HX[s], [lanebase + (ms[u] & 255)], ones)
                return 0

            lax.fori_loop(0, NVEC // 4, tbody, 0)

        for t in range(4):
            cur = HX if t % 2 == 0 else HY
            nxt = HY if t % 2 == 0 else HX
            src = A if t % 2 == 0 else B
            dst = B if t % 2 == 0 else A

            if t > 0:
                def hbody(j, _):
                    i0 = 2 * j
                    ks = [src[s][pl.ds((i0 + u) * L, L)]
                          for u in range(2) for s in range(4)]
                    idxs = []
                    for k in ks:
                        d = lax.shift_right_logical(k, 8 * t)
                        if t < 3:
                            d = d & 255
                        idxs.append(lanebase + d)
                    for n, hidx in enumerate(idxs):
                        plsc.addupdate_scatter(cur[n % 4], [hidx], ones)
                    return 0

                lax.fori_loop(0, NVEC // 2, hbody, 0)

            def l1(g, _):
                sums = [None] * 4
                for l in range(L):
                    vs = [cur[s][pl.ds(l * RADIX + g * L, L)] for s in range(4)]
                    for s in range(4):
                        sums[s] = vs[s] if l == 0 else sums[s] + vs[s]
                for s in range(4):
                    TOT[s][pl.ds(g * L, L)] = sums[s]
                return 0

            lax.fori_loop(0, RADIX // L, l1, 0)

            def l2(g, carry):
                nc = []
                for s in range(4):
                    v = TOT[s][pl.ds(g * L, L)]
                    sc = plsc.cumsum(v)
                    DB[s][pl.ds(g * L, L)] = (sc - v) + carry[s]
                    nc.append(carry[s] + jnp.sum(v))
                return tuple(nc)

            lax.fori_loop(0, RADIX // L, l2, (jnp.int32(0),) * 4)

            def l3(g, _):
                runs = [DB[s][pl.ds(g * L, L)] for s in range(4)]
                for l in range(L):
                    vs = [cur[s][pl.ds(l * RADIX + g * L, L)] for s in range(4)]
                    for s in range(4):
                        cur[s][pl.ds(l * RADIX + g * L, L)] = runs[s]
                        nxt[s][pl.ds(l * RADIX + g * L, L)] = zi
                    for s in range(4):
                        runs[s] = runs[s] + vs[s]
                return 0

            lax.fori_loop(0, RADIX // L, l3, 0)

            def pbody(i, _):
                ks = [src[s][pl.ds(i * L, L)] for s in range(4)]
                hidxs = []
                for k in ks:
                    if t == 0:
                        d = k & 255
                    elif t == 3:
                        d = lax.shift_right_logical(k, 24)
                    else:
                        d = lax.shift_right_logical(k, 8 * t) & 255
                    hidxs.append(lanebase + d)
                rs = [plsc.load_gather(cur[s], [hidxs[s]]) for s in range(4)]
                for s in range(4):
                    plsc.store_scatter(cur[s], [hidxs[s]], rs[s] + ones)
                for s in range(4):
                    pos = ((rs[s] & (CHUNK - 1)) << 4) | (rs[s] >> 9)
                    plsc.store_scatter(dst[s], [pos], ks[s])
                return 0

            lax.fori_loop(0, NVEC, pbody, 0, unroll=2)

        def dbody(i, _):
            ys = [A[s][pl.ds((2 * i + par) * L, L)]
                  for par in range(2) for s in range(4)]
            fs = []
            for y in ys:
                x = y ^ (MININT | ~(y >> 31))
                fs.append(lax.bitcast_convert_type(x, jnp.float32))
            for par in range(2):
                for pi in range(2):
                    fa = fs[par * 4 + 2 * pi]
                    fb = fs[par * 4 + 2 * pi + 1]
                    ACC[2 * pi + par][...] = ACC[2 * pi + par][...] + jnp.abs(fa - fb)
            return 0

        lax.fori_loop(0, NVEC // 2, dbody, 0)
        return 0

    lax.fori_loop(0, CPW // 2, pair_step, 0)
    acc0[...] = (ACC[0][...] + ACC[1][...]) + (ACC[2][...] + ACC[3][...])
    pltpu.sync_copy(acc0, out_hbm.at[wid])


_sc_wasserstein = functools.partial(
    pl.kernel,
    out_type=jax.ShapeDtypeStruct((NW, L), jnp.float32),
    mesh=plsc.VectorSubcoreMesh(core_axis_name="c", subcore_axis_name="s"),
    compiler_params=pltpu.CompilerParams(needs_layout_passes=False),
    scratch_types=(
        [pltpu.VMEM((NT,), jnp.float32) for _ in range(2)]
        + [pltpu.VMEM((NT,), jnp.int32) for _ in range(8)]
        + [pltpu.VMEM((HIST,), jnp.int32) for _ in range(8)]
        + [pltpu.VMEM((RADIX,), jnp.int32) for _ in range(8)]
        + [pltpu.VMEM((L,), jnp.float32) for _ in range(4)]
        + [pltpu.SemaphoreType.DMA for _ in range(2)]
    ),
)(_sc_body)


def kernel(pred_waveforms, obs_waveforms):
    nt, ntr, ch = pred_waveforms.shape
    pred_t = _transpose(pred_waveforms.reshape(nt, ntr * ch))
    obs_t = _transpose(obs_waveforms.reshape(nt, ntr * ch))
    partials = _sc_wasserstein(pred_t, obs_t)
    return jnp.sum(partials) / (nt * ntr * ch)


# speedup vs baseline: 5.6003x; 1.2718x over previous
import functools

import jax
import jax.numpy as jnp
import numpy as np
from jax import lax
from jax.experimental import pallas as pl
from jax.experimental.pallas import tpu as pltpu
from jax.experimental.pallas import tpu_sc as plsc

NC, NS, L = 2, 16, 16
NW = NC * NS
NT = 8192
NCOL = 1536
CPW = NCOL // NW
CHUNK = NT // L
NVEC = NT // L
RADIX = 256
HIST = RADIX * L
MININT = np.int32(-2147483648)


def _transpose_body(x_ref, o_ref):
    o_ref[...] = lax.bitcast_convert_type(x_ref[...].T, jnp.int32)


def _transpose(x):
    bt, bc = 1024, 512
    return pl.pallas_call(
        _transpose_body,
        grid=(NT // bt, NCOL // bc),
        in_specs=[pl.BlockSpec((bt, bc), lambda i, j: (i, j))],
        out_specs=pl.BlockSpec((bc, bt), lambda i, j: (j, i)),
        out_shape=jax.ShapeDtypeStruct((NCOL, NT), jnp.int32),
    )(x)


def _sc_body(pred_hbm, obs_hbm, out_hbm,
             a0, a1, a2, a3, b0, b1, b2, b3,
             hx0, hx1, hx2, hx3, hy0, hy1, hy2, hy3,
             tt0, tt1, tt2, tt3, dg0, dg1, dg2, dg3,
             stage, sem):
    cid = lax.axis_index("c")
    sid = lax.axis_index("s")
    wid = sid * NC + cid
    lane = lax.iota(jnp.int32, L)
    ones = jnp.ones((L,), jnp.int32)
    zi = jnp.zeros((L,), jnp.int32)
    fz = jnp.zeros((L,), jnp.float32)
    lanebase = lane * RADIX
    A = [a0, a1, a2, a3]
    B = [b0, b1, b2, b3]
    HX = [hx0, hx1, hx2, hx3]
    HY = [hy0, hy1, hy2, hy3]
    TOT = [tt0, tt1, tt2, tt3]
    DB = [dg0, dg1, dg2, dg3]

    def dmad(s, c0):
        ref = pred_hbm if s % 2 == 0 else obs_hbm
        return pltpu.make_async_copy(ref.at[c0 + s // 2], A[s], sem)

    def zinit(i, _):
        for s in range(4):
            HX[s][pl.ds(i * L, L)] = zi
        return 0

    lax.fori_loop(0, HIST // L, zinit, 0)

    base0 = wid * CPW
    for s in range(4):
        dmad(s, base0).start()

    def half_step(c0, cnext, HA, HB, accs):
        for s in range(4):
            dmad(s, c0).wait()

        @plsc.parallel_loop(0, NVEC, unroll=2)
        def h0(i):
            ys = [A[s][pl.ds(i * L, L)] for s in range(4)]
            for s in range(4):
                m = ys[s] ^ (MININT | (ys[s] >> 31))
                d = lax.shift_right_logical(m, 8) & 255
                plsc.addupdate_scatter(HA[s], [lanebase + d], ones)

        for t in range(3):
            cur = HA if t % 2 == 0 else HB
            nxt = HB if t % 2 == 0 else HA
            src = A if t % 2 == 0 else B
            dst = B if t % 2 == 0 else A

            @plsc.parallel_loop(0, RADIX // L)
            def l1(g):
                sums = [None] * 4
                for l in range(L):
                    vs = [cur[s][pl.ds(l * RADIX + g * L, L)] for s in range(4)]
                    for s in range(4):
                        sums[s] = vs[s] if l == 0 else sums[s] + vs[s]
                for s in range(4):
                    TOT[s][pl.ds(g * L, L)] = sums[s]

            def l2(g, carry):
                nc = []
                for s in range(4):
                    v = TOT[s][pl.ds(g * L, L)]
                    sc = plsc.cumsum(v)
                    DB[s][pl.ds(g * L, L)] = (sc - v) + carry[s]
                    nc.append(carry[s] + jnp.sum(v))
                return tuple(nc)

            plsc.parallel_loop(0, RADIX // L, carry=(jnp.int32(0),) * 4)(l2)

            @plsc.parallel_loop(0, RADIX // L)
            def l3(g):
                runs = [DB[s][pl.ds(g * L, L)] for s in range(4)]
                for l in range(L):
                    vs = [cur[s][pl.ds(l * RADIX + g * L, L)] for s in range(4)]
                    for s in range(4):
                        cur[s][pl.ds(l * RADIX + g * L, L)] = runs[s]
                        nxt[s][pl.ds(l * RADIX + g * L, L)] = zi
                    for s in range(4):
                        runs[s] = runs[s] + vs[s]

            def pb(i, _):
                ks = [src[s][pl.ds(i * L, L)] for s in range(4)]
                if t == 0:
                    outs = [k ^ (MININT | (k >> 31)) for k in ks]
                    digs = [lax.shift_right_logical(m, 8) & 255 for m in outs]
                elif t == 1:
                    outs = ks
                    digs = [lax.shift_right_logical(k, 16) & 255 for k in ks]
                else:
                    outs = ks
                    digs = [lax.shift_right_logical(k, 24) for k in ks]
                hidxs = [lanebase + d for d in digs]
                rs = [plsc.load_gather(cur[s], [hidxs[s]]) for s in range(4)]
                for s in range(4):
                    plsc.store_scatter(cur[s], [hidxs[s]], rs[s] + ones)
                for s in range(4):
                    pos = ((rs[s] & (CHUNK - 1)) << 4) | (rs[s] >> 9)
                    plsc.store_scatter(dst[s], [pos], outs[s])
                return 0

            lax.fori_loop(0, NVEC, pb, 0, unroll=2)

            if t < 2:
                sh2 = 16 if t == 0 else 24

                @plsc.parallel_loop(0, NVEC, unroll=2)
                def hnx(i):
                    ks = [dst[s][pl.ds(i * L, L)] for s in range(4)]
                    for s in range(4):
                        d = lax.shift_right_logical(ks[s], sh2)
                        if sh2 < 24:
                            d = d & 255
                        plsc.addupdate_scatter(nxt[s], [lanebase + d], ones)

        for s in range(4):
            dmad(s, cnext).start()

        def dbody(i, acc):
            ys = [B[s][pl.ds((2 * i + par) * L, L)]
                  for par in range(2) for s in range(4)]
            fs = []
            for y in ys:
                x = y ^ (MININT | ~(y >> 31))
                fs.append(lax.bitcast_convert_type(x, jnp.float32))
            na = list(acc)
            for par in range(2):
                for pi in range(2):
                    fa = fs[par * 4 + 2 * pi]
                    fb = fs[par * 4 + 2 * pi + 1]
                    na[2 * pi + par] = na[2 * pi + par] + jnp.abs(fa - fb)
            return tuple(na)

        return plsc.parallel_loop(0, NVEC // 2, carry=accs)(dbody)

    def outer(q, accs):
        c0 = base0 + 4 * q
        accs = half_step(c0, c0 + 2, HX, HY, accs)
        cnext = jnp.minimum(c0 + 4, NCOL - 2)
        accs = half_step(c0 + 2, cnext, HY, HX, accs)
        return accs

    accs = lax.fori_loop(0, CPW // 4, outer, (fz, fz, fz, fz))
    for s in range(4):
        dmad(s, 0).wait()
    stage[...] = (accs[0] + accs[1]) + (accs[2] + accs[3])
    pltpu.sync_copy(stage, out_hbm.at[wid])


_sc_wasserstein = functools.partial(
    pl.kernel,
    out_type=jax.ShapeDtypeStruct((NW, L), jnp.float32),
    mesh=plsc.VectorSubcoreMesh(core_axis_name="c", subcore_axis_name="s"),
    compiler_params=pltpu.CompilerParams(needs_layout_passes=False),
    scratch_types=(
        [pltpu.VMEM((NT,), jnp.int32) for _ in range(8)]
        + [pltpu.VMEM((HIST,), jnp.int32) for _ in range(8)]
        + [pltpu.VMEM((RADIX,), jnp.int32) for _ in range(8)]
        + [pltpu.VMEM((L,), jnp.float32)]
        + [pltpu.SemaphoreType.DMA]
    ),
)(_sc_body)


def kernel(pred_waveforms, obs_waveforms):
    nt, ntr, ch = pred_waveforms.shape
    pred_t = _transpose(pred_waveforms.reshape(nt, ntr * ch))
    obs_t = _transpose(obs_waveforms.reshape(nt, ntr * ch))
    partials = _sc_wasserstein(pred_t, obs_t)
    return jnp.sum(partials) / (nt * ntr * ch)
